# baseline (speedup 1.0000x reference)
import jax
import jax.numpy as jnp
from jax.experimental import pallas as pl
from jax.experimental.pallas import tpu as pltpu

_BN = 512
_KP = 2


def _body(x_hbm, w_ref, b_ref, o_ref, xv_ref, sems):
    j = pl.program_id(0)
    k = pl.program_id(1)
    kw = x_hbm.shape[1] // _KP

    @pl.when(jnp.logical_and(j == 0, k == 0))
    def _start_copies():
        for c in range(_KP):
            pltpu.make_async_copy(
                x_hbm.at[:, c * kw:(c + 1) * kw],
                xv_ref.at[:, c * kw:(c + 1) * kw],
                sems.at[c],
            ).start()

    for c in range(_KP):
        @pl.when(jnp.logical_and(j == 0, k == c))
        def _wait_chunk(c=c):
            pltpu.make_async_copy(
                x_hbm.at[:, c * kw:(c + 1) * kw],
                xv_ref.at[:, c * kw:(c + 1) * kw],
                sems.at[c],
            ).wait()

    part = jax.lax.dot_general(
        xv_ref[:, pl.ds(k * kw, kw)], w_ref[...],
        (((1,), (1,)), ((), ())),
        preferred_element_type=jnp.float32,
    )

    @pl.when(k == 0)
    def _first():
        o_ref[...] = part

    @pl.when(k == _KP - 1)
    def _last():
        o_ref[...] = o_ref[...] + part + b_ref[...]


def kernel(x, weight, bias):
    if x.ndim == 4:
        x = x.reshape(x.shape[0], -1)
    M, K = x.shape
    N = weight.shape[0]
    bias2 = bias.reshape(1, N)
    bk = K // _KP
    grid = (N // _BN, _KP)
    return pl.pallas_call(
        _body,
        grid=grid,
        in_specs=[
            pl.BlockSpec(memory_space=pl.ANY),
            pl.BlockSpec((_BN, bk), lambda j, k: (j, k)),
            pl.BlockSpec((1, _BN), lambda j, k: (0, j)),
        ],
        out_specs=pl.BlockSpec((M, _BN), lambda j, k: (0, j)),
        out_shape=jax.ShapeDtypeStruct((M, N), jnp.float32),
        scratch_shapes=[
            pltpu.VMEM((M, K), jnp.float32),
            pltpu.SemaphoreType.DMA((_KP,)),
        ],
        compiler_params=pltpu.CompilerParams(
            dimension_semantics=("arbitrary", "arbitrary"),
            vmem_limit_bytes=64 * 1024 * 1024,
        ),
    )(x, weight, bias2)
